# baseline (speedup 1.0000x reference)
import jax
import jax.numpy as jnp
from jax import lax
from jax.experimental import pallas as pl
from jax.experimental.pallas import tpu as pltpu
from jax.experimental.pallas import tpu_sc as plsc

_R, _N, _K, _D, _H = 16, 4096, 128, 128, 128
_NEG = -1000000000.0
_LANES = 16
_KH = _K // 2


def _sc_body(x_hbm, maskflat_hbm, cidx_hbm, rows_out, mbits_out,
             cidx_v, idx_v, mb_v, rows_v, sem1, sem2):
    wid = lax.axis_index("s") * 2 + lax.axis_index("c")
    robot = wid // 2
    half = wid % 2

    pltpu.sync_copy(cidx_hbm.at[robot], cidx_v)
    base = robot * _N
    off = half * _KH
    for c in range(_KH // _LANES):
        ci = cidx_v[pl.ds(off + c * _LANES, _LANES)]
        idx_v[pl.ds(c * _LANES, _LANES)] = ci + base

    g_rows = pltpu.async_copy(x_hbm.at[idx_v], rows_v, sem1)
    g_bits = pltpu.async_copy(maskflat_hbm.at[idx_v], mb_v, sem2)
    g_rows.wait()
    g_bits.wait()

    pltpu.sync_copy(rows_v, rows_out.at[pl.ds(wid * _KH, _KH)])
    pltpu.sync_copy(mb_v, mbits_out.at[wid])


def _sc_gather(x_flat, maskflat, cand_idx):
    mesh = plsc.VectorSubcoreMesh(core_axis_name="c", subcore_axis_name="s")
    f = pl.kernel(
        _sc_body,
        mesh=mesh,
        compiler_params=pltpu.CompilerParams(needs_layout_passes=False),
        out_type=[
            jax.ShapeDtypeStruct((_R * _K, _D), jnp.float32),
            jax.ShapeDtypeStruct((2 * _R, _KH), jnp.int32),
        ],
        scratch_types=[
            pltpu.VMEM((_K,), jnp.int32),
            pltpu.VMEM((_KH,), jnp.int32),
            pltpu.VMEM((_KH,), jnp.int32),
            pltpu.VMEM((_KH, _D), jnp.float32),
            pltpu.SemaphoreType.DMA,
            pltpu.SemaphoreType.DMA,
        ],
    )
    return f(x_flat, maskflat, cand_idx)


def _tc_body(maskf_ref, x_ref, rows_ref, mbits_ref, cidx_ref, cmask_ref,
             wea_ref, bea_ref, wec_ref, bec_ref, lng_ref, lnb_ref,
             wa_ref, ba_ref, watt_ref, batt_ref,
             wc1_ref, bc1_ref, wc2_ref, bc2_ref,
             logits_ref, v_ref, sx_ref):
    i = pl.program_id(0)

    m = maskf_ref[pl.ds(i, 1), :]
    nq = _N // 4
    parts = [
        jnp.dot(m[:, q * nq:(q + 1) * nq], x_ref[0][q * nq:(q + 1) * nq, :],
                preferred_element_type=jnp.float32)
        for q in range(4)
    ]
    sx_ref[pl.ds(i, 1), :] = (parts[0] + parts[1]) + (parts[2] + parts[3])

    n_i = jnp.sum(m)
    empty = n_i == 0.0
    rows = rows_ref[...] * jnp.where(empty, 0.0, 1.0)
    h = jnp.dot(rows, wea_ref[...],
                preferred_element_type=jnp.float32) + bea_ref[...]
    mu = jnp.mean(h, axis=-1, keepdims=True)
    d = h - mu
    var = jnp.mean(d * d, axis=-1, keepdims=True)
    hn = d * lax.rsqrt(var + 1e-5) * lng_ref[...] + lnb_ref[...]
    sc = lax.dot_general(wa_ref[...], hn, (((1,), (1,)), ((), ())),
                         preferred_element_type=jnp.float32)
    vals = jnp.tanh(sc + ba_ref[...]) * 5.0

    ci_row = cidx_ref[pl.ds(i, 1), :]
    cm_row = cmask_ref[pl.ds(i, 1), :]
    mb_row = mbits_ref[pl.ds(i, 1), :]
    cif = (ci_row == 0).astype(jnp.float32)
    mbf = (mb_row > 0).astype(jnp.float32)
    cmf = (cm_row > 0).astype(jnp.float32)
    vf = cmf * jnp.where(empty, cif, mbf)

    incl = vf
    for s in (1, 2, 4, 8, 16, 32, 64):
        incl = incl + jnp.concatenate(
            [jnp.zeros((1, s), jnp.float32), incl[:, :_K - s]], axis=1)
    excl = incl - vf
    kiota = lax.broadcasted_iota(jnp.int32, (_K, _K), 0)
    jiota = lax.broadcasted_iota(jnp.int32, (_K, _K), 1)
    eye = (kiota == jiota).astype(jnp.float32)
    both = jnp.concatenate([excl, vf], axis=0)
    cols = lax.dot_general(eye, both, (((1,), (1,)), ((), ())),
                           preferred_element_type=jnp.float32)
    onehot = ((cols[:, 0:1] == jiota.astype(jnp.float32)) & (cols[:, 1:2] > 0)
              ).astype(jnp.float32)
    compacted = lax.dot_general(vals, onehot, (((1,), (0,)), ((), ())),
                                preferred_element_type=jnp.float32)
    nv = jnp.sum(vf)
    lane = lax.broadcasted_iota(jnp.int32, (1, _K), 1).astype(jnp.float32)
    logits_ref[pl.ds(i, 1), :] = jnp.where(lane < nv, compacted, _NEG)

    @pl.when(i == _R - 1)
    def _():
        nvec = jnp.sum(maskf_ref[...], axis=1, keepdims=True)
        denom = jnp.maximum(nvec, 1.0)
        emb = jnp.dot(sx_ref[...] / denom, wec_ref[...],
                      preferred_element_type=jnp.float32) + bec_ref[...]
        a = lax.dot_general(watt_ref[...], emb, (((1,), (1,)), ((), ())),
                            preferred_element_type=jnp.float32) + batt_ref[...]
        a = a - jnp.max(a, axis=-1, keepdims=True)
        e = jnp.exp(a)
        w = e / jnp.sum(e, axis=-1, keepdims=True)
        g = jnp.dot(w, emb, preferred_element_type=jnp.float32)
        hmid = jnp.maximum(
            jnp.dot(g, wc1_ref[...],
                    preferred_element_type=jnp.float32) + bc1_ref[...], 0.0)
        v_ref[...] = (jnp.sum(hmid * wc2_ref[...], axis=-1, keepdims=True)
                      + bc2_ref[...])


def _tc_head(maskf, x, rows, mbits, cidx, cmask,
             wea, bea, wec, bec, lng, lnb,
             wa, ba, watt, batt, wc1, bc1, wc2, bc2):
    full = lambda shape: pl.BlockSpec(shape, lambda i: tuple(0 for _ in shape))
    return pl.pallas_call(
        _tc_body,
        grid=(_R,),
        in_specs=[
            full((_R, _N)),
            pl.BlockSpec((1, _N, _D), lambda i: (i, 0, 0)),
            pl.BlockSpec((_K, _D), lambda i: (i, 0)),
            full((_R, _K)),
            full((_R, _K)),
            full((_R, _K)),
            full((_D, _H)), full((1, _H)),
            full((_D, _H)), full((1, _H)),
            full((1, _H)), full((1, _H)),
            full((1, _H)), full((1, 1)),
            full((1, _H)), full((1, 1)),
            full((_H, _H)), full((1, _H)),
            full((1, _H)), full((1, 1)),
        ],
        out_specs=[
            pl.BlockSpec((_R, _K), lambda i: (0, 0)),
            pl.BlockSpec((1, 1), lambda i: (0, 0)),
        ],
        out_shape=[
            jax.ShapeDtypeStruct((_R, _K), jnp.float32),
            jax.ShapeDtypeStruct((1, 1), jnp.float32),
        ],
        scratch_shapes=[pltpu.VMEM((_R, _D), jnp.float32)],
    )(maskf, x, rows, mbits, cidx, cmask, wea, bea, wec, bec, lng, lnb,
      wa, ba, watt, batt, wc1, bc1, wc2, bc2)


def kernel(x, node_mask, cand_idx, cand_mask,
           W_enc_a, b_enc_a, W_enc_c, b_enc_c, ln_g, ln_b,
           W_actor, b_actor, W_attn, b_attn, W_c1, b_c1, W_c2, b_c2):
    maskf = node_mask.astype(jnp.float32)
    maskflat = node_mask.astype(jnp.int32).reshape(_R * _N)
    cmask_i32 = cand_mask.astype(jnp.int32)
    x_flat = x.reshape(_R * _N, _D)

    rows, mbits_h = _sc_gather(x_flat, maskflat, cand_idx)
    mbits = mbits_h.reshape(_R, _K)

    logits, v = _tc_head(
        maskf, x, rows, mbits, cand_idx, cmask_i32,
        W_enc_a, b_enc_a.reshape(1, _H),
        W_enc_c, b_enc_c.reshape(1, _H),
        ln_g.reshape(1, _H), ln_b.reshape(1, _H),
        W_actor.reshape(1, _H), b_actor.reshape(1, 1),
        W_attn.reshape(1, _H), b_attn.reshape(1, 1),
        W_c1, b_c1.reshape(1, _H),
        W_c2.reshape(1, _H), b_c2.reshape(1, 1),
    )
    return logits, v.reshape(())


# speedup vs baseline: 1.0490x; 1.0490x over previous
import jax
import jax.numpy as jnp
from jax import lax
from jax.experimental import pallas as pl
from jax.experimental.pallas import tpu as pltpu
from jax.experimental.pallas import tpu_sc as plsc

_R, _N, _K, _D, _H = 16, 4096, 128, 128, 128
_NEG = -1000000000.0
_LANES = 16
_KH = _K // 2


def _sc_body(x_hbm, maskflat_hbm, cidx_hbm, rows_out, mbits_out,
             cidx_v, idx_v, mb_v, rows_v, sem1, sem2):
    wid = lax.axis_index("s") * 2 + lax.axis_index("c")
    robot = wid // 2
    half = wid % 2

    pltpu.sync_copy(cidx_hbm.at[robot], cidx_v)
    base = robot * _N
    off = half * _KH
    for c in range(_KH // _LANES):
        ci = cidx_v[pl.ds(off + c * _LANES, _LANES)]
        idx_v[pl.ds(c * _LANES, _LANES)] = ci + base

    g_rows = pltpu.async_copy(x_hbm.at[idx_v], rows_v, sem1)
    g_bits = pltpu.async_copy(maskflat_hbm.at[idx_v], mb_v, sem2)
    g_rows.wait()
    g_bits.wait()

    pltpu.sync_copy(rows_v, rows_out.at[pl.ds(wid * _KH, _KH)])
    pltpu.sync_copy(mb_v, mbits_out.at[wid])


def _sc_gather(x_flat, maskflat, cand_idx):
    mesh = plsc.VectorSubcoreMesh(core_axis_name="c", subcore_axis_name="s")
    f = pl.kernel(
        _sc_body,
        mesh=mesh,
        compiler_params=pltpu.CompilerParams(needs_layout_passes=False),
        out_type=[
            jax.ShapeDtypeStruct((_R * _K, _D), jnp.float32),
            jax.ShapeDtypeStruct((2 * _R, _KH), jnp.int32),
        ],
        scratch_types=[
            pltpu.VMEM((_K,), jnp.int32),
            pltpu.VMEM((_KH,), jnp.int32),
            pltpu.VMEM((_KH,), jnp.int32),
            pltpu.VMEM((_KH, _D), jnp.float32),
            pltpu.SemaphoreType.DMA,
            pltpu.SemaphoreType.DMA,
        ],
    )
    return f(x_flat, maskflat, cand_idx)


def _tc_body(maskf_ref, x_ref, rows_ref, mbits_ref, cidx_ref, cmask_ref,
             wea_ref, bea_ref, wec_ref, bec_ref, lng_ref, lnb_ref,
             wa_ref, ba_ref, watt_ref, batt_ref,
             wc1_ref, bc1_ref, wc2_ref, bc2_ref,
             logits_ref, v_ref, sx_ref):
    i = pl.program_id(0)

    m = maskf_ref[pl.ds(i, 1), :]
    nq = _N // 4
    parts = [
        jnp.dot(m[:, q * nq:(q + 1) * nq], x_ref[0][q * nq:(q + 1) * nq, :],
                preferred_element_type=jnp.float32)
        for q in range(4)
    ]
    sx_ref[pl.ds(i, 1), :] = (parts[0] + parts[1]) + (parts[2] + parts[3])

    n_i = jnp.sum(m)
    empty = n_i == 0.0
    rows = rows_ref[...] * jnp.where(empty, 0.0, 1.0)
    h = jnp.dot(rows, wea_ref[...],
                preferred_element_type=jnp.float32) + bea_ref[...]
    mu = jnp.mean(h, axis=-1, keepdims=True)
    d = h - mu
    var = jnp.mean(d * d, axis=-1, keepdims=True)
    hn = d * lax.rsqrt(var + 1e-5) * lng_ref[...] + lnb_ref[...]
    sc = lax.dot_general(wa_ref[...], hn, (((1,), (1,)), ((), ())),
                         preferred_element_type=jnp.float32)
    vals = jnp.tanh(sc + ba_ref[...]) * 5.0

    ci_row = cidx_ref[pl.ds(i, 1), :]
    cm_row = cmask_ref[pl.ds(i, 1), :]
    mb_row = mbits_ref[pl.ds(i, 1), :]
    cif = (ci_row == 0).astype(jnp.float32)
    mbf = (mb_row > 0).astype(jnp.float32)
    cmf = (cm_row > 0).astype(jnp.float32)
    vf = cmf * jnp.where(empty, cif, mbf)

    kiota = lax.broadcasted_iota(jnp.int32, (_K, _K), 0)
    jiota = lax.broadcasted_iota(jnp.int32, (_K, _K), 1)
    vb = jnp.broadcast_to(vf, (_K, _K))
    excl = jnp.sum(jnp.where(jiota < kiota, vb, 0.0), axis=1, keepdims=True)
    vcol = jnp.sum(jnp.where(jiota == kiota, vb, 0.0), axis=1, keepdims=True)
    onehot = ((excl == jiota.astype(jnp.float32)) & (vcol > 0)
              ).astype(jnp.float32)
    compacted = lax.dot_general(vals, onehot, (((1,), (0,)), ((), ())),
                                preferred_element_type=jnp.float32)
    nv = jnp.sum(vf)
    lane = lax.broadcasted_iota(jnp.int32, (1, _K), 1).astype(jnp.float32)
    logits_ref[pl.ds(i, 1), :] = jnp.where(lane < nv, compacted, _NEG)

    @pl.when(i == _R - 1)
    def _():
        nvec = jnp.sum(maskf_ref[...], axis=1, keepdims=True)
        denom = jnp.maximum(nvec, 1.0)
        emb = jnp.dot(sx_ref[...] / denom, wec_ref[...],
                      preferred_element_type=jnp.float32) + bec_ref[...]
        a = lax.dot_general(watt_ref[...], emb, (((1,), (1,)), ((), ())),
                            preferred_element_type=jnp.float32) + batt_ref[...]
        a = a - jnp.max(a, axis=-1, keepdims=True)
        e = jnp.exp(a)
        w = e / jnp.sum(e, axis=-1, keepdims=True)
        g = jnp.dot(w, emb, preferred_element_type=jnp.float32)
        hmid = jnp.maximum(
            jnp.dot(g, wc1_ref[...],
                    preferred_element_type=jnp.float32) + bc1_ref[...], 0.0)
        v_ref[...] = (jnp.sum(hmid * wc2_ref[...], axis=-1, keepdims=True)
                      + bc2_ref[...])


def _tc_head(maskf, x, rows, mbits, cidx, cmask,
             wea, bea, wec, bec, lng, lnb,
             wa, ba, watt, batt, wc1, bc1, wc2, bc2):
    full = lambda shape: pl.BlockSpec(shape, lambda i: tuple(0 for _ in shape))
    return pl.pallas_call(
        _tc_body,
        grid=(_R,),
        in_specs=[
            full((_R, _N)),
            pl.BlockSpec((1, _N, _D), lambda i: (i, 0, 0)),
            pl.BlockSpec((_K, _D), lambda i: (i, 0)),
            full((_R, _K)),
            full((_R, _K)),
            full((_R, _K)),
            full((_D, _H)), full((1, _H)),
            full((_D, _H)), full((1, _H)),
            full((1, _H)), full((1, _H)),
            full((1, _H)), full((1, 1)),
            full((1, _H)), full((1, 1)),
            full((_H, _H)), full((1, _H)),
            full((1, _H)), full((1, 1)),
        ],
        out_specs=[
            pl.BlockSpec((_R, _K), lambda i: (0, 0)),
            pl.BlockSpec((1, 1), lambda i: (0, 0)),
        ],
        out_shape=[
            jax.ShapeDtypeStruct((_R, _K), jnp.float32),
            jax.ShapeDtypeStruct((1, 1), jnp.float32),
        ],
        scratch_shapes=[pltpu.VMEM((_R, _D), jnp.float32)],
    )(maskf, x, rows, mbits, cidx, cmask, wea, bea, wec, bec, lng, lnb,
      wa, ba, watt, batt, wc1, bc1, wc2, bc2)


def kernel(x, node_mask, cand_idx, cand_mask,
           W_enc_a, b_enc_a, W_enc_c, b_enc_c, ln_g, ln_b,
           W_actor, b_actor, W_attn, b_attn, W_c1, b_c1, W_c2, b_c2):
    maskf = node_mask.astype(jnp.float32)
    maskflat = node_mask.astype(jnp.int32).reshape(_R * _N)
    cmask_i32 = cand_mask.astype(jnp.int32)
    x_flat = x.reshape(_R * _N, _D)

    rows, mbits_h = _sc_gather(x_flat, maskflat, cand_idx)
    mbits = mbits_h.reshape(_R, _K)

    logits, v = _tc_head(
        maskf, x, rows, mbits, cand_idx, cmask_i32,
        W_enc_a, b_enc_a.reshape(1, _H),
        W_enc_c, b_enc_c.reshape(1, _H),
        ln_g.reshape(1, _H), ln_b.reshape(1, _H),
        W_actor.reshape(1, _H), b_actor.reshape(1, 1),
        W_attn.reshape(1, _H), b_attn.reshape(1, 1),
        W_c1, b_c1.reshape(1, _H),
        W_c2.reshape(1, _H), b_c2.reshape(1, 1),
    )
    return logits, v.reshape(())
